# Initial kernel scaffold; baseline (speedup 1.0000x reference)
import functools

import jax
import jax.numpy as jnp
from jax import lax
from jax.experimental import pallas as pl
from jax.experimental.pallas import tpu as pltpu
from jax.experimental.pallas import tpu_sc as plsc

N = 10000
D = 128
E = 320000
NC = 2
NS = 16
TPT = E // NC // NS
CW = 80
NCH = TPT // CW
NG = CW // 16
RPT = N // NS
DNR = 80

_BM = 1000


def _attn_tail(feat, al, ar, feat_ref, elr_ref):
    feat_ref[...] = feat
    el = jnp.sum(feat * al, axis=1)
    er = jnp.sum(feat * ar, axis=1)
    elr_ref[0:1, :] = el[None, :]
    elr_ref[1:2, :] = er[None, :]


def _tc_front_body(x_ref, w_ref, al_ref, ar_ref, feat_ref, elr_ref):
    feat = jnp.dot(x_ref[...], w_ref[...], preferred_element_type=jnp.float32)
    _attn_tail(feat, al_ref[...], ar_ref[...], feat_ref, elr_ref)


def _tc_mid_body(p_ref, b_ref, w_ref, al_ref, ar_ref, feat_ref, elr_ref):
    x = jax.nn.relu(p_ref[0] + p_ref[1] + b_ref[...])
    feat = jnp.dot(x, w_ref[...], preferred_element_type=jnp.float32)
    _attn_tail(feat, al_ref[...], ar_ref[...], feat_ref, elr_ref)


def _tc_head_body(p_ref, b_ref, wlin_ref, blin_ref, df_ref, sc_ref, res_ref):
    x = jax.nn.relu(p_ref[0] + p_ref[1] + b_ref[...])
    z = jnp.dot(x, wlin_ref[...], preferred_element_type=jnp.float32)
    hout = jax.nn.sigmoid(z + blin_ref[...])
    hT = hout.T
    a0 = sc_ref[0]
    b0 = sc_ref[1]
    g0 = sc_ref[2]
    dtv = sc_ref[3]
    beta = hT[0:1, :] * b0
    gamma = hT[1:2, :] * g0
    alphas = hT[2:3, :] * a0
    dfT = df_ref[...].T
    us = dfT[0:1, :]
    s = dfT[1:2, :]
    res_ref[0:1, :] = us + (alphas - beta * us) * dtv
    res_ref[1:2, :] = s + (beta * us - gamma * s) * dtv
    res_ref[2:3, :] = alphas
    res_ref[3:4, :] = beta
    res_ref[4:5, :] = gamma
    res_ref[5:8, :] = jnp.zeros((3, res_ref.shape[1]), jnp.float32)


def _tc_front(x, w, al, ar):
    k = x.shape[1]
    grid = N // _BM
    return pl.pallas_call(
        _tc_front_body,
        grid=(grid,),
        in_specs=[
            pl.BlockSpec((_BM, k), lambda i: (i, 0)),
            pl.BlockSpec((k, D), lambda i: (0, 0)),
            pl.BlockSpec((1, D), lambda i: (0, 0)),
            pl.BlockSpec((1, D), lambda i: (0, 0)),
        ],
        out_specs=[
            pl.BlockSpec((_BM, D), lambda i: (i, 0)),
            pl.BlockSpec((2, _BM), lambda i: (0, i)),
        ],
        out_shape=[
            jax.ShapeDtypeStruct((N, D), jnp.float32),
            jax.ShapeDtypeStruct((2, N), jnp.float32),
        ],
    )(x, w, al, ar)


def _tc_mid(p, b, w, al, ar):
    grid = N // _BM
    return pl.pallas_call(
        _tc_mid_body,
        grid=(grid,),
        in_specs=[
            pl.BlockSpec((2, _BM, D), lambda i: (0, i, 0)),
            pl.BlockSpec((1, D), lambda i: (0, 0)),
            pl.BlockSpec((D, D), lambda i: (0, 0)),
            pl.BlockSpec((1, D), lambda i: (0, 0)),
            pl.BlockSpec((1, D), lambda i: (0, 0)),
        ],
        out_specs=[
            pl.BlockSpec((_BM, D), lambda i: (i, 0)),
            pl.BlockSpec((2, _BM), lambda i: (0, i)),
        ],
        out_shape=[
            jax.ShapeDtypeStruct((N, D), jnp.float32),
            jax.ShapeDtypeStruct((2, N), jnp.float32),
        ],
    )(p, b, w, al, ar)


def _tc_head(p, b, wlin, blin, df, scal):
    grid = N // _BM
    return pl.pallas_call(
        _tc_head_body,
        grid=(grid,),
        in_specs=[
            pl.BlockSpec((2, _BM, D), lambda i: (0, i, 0)),
            pl.BlockSpec((1, D), lambda i: (0, 0)),
            pl.BlockSpec((D, D), lambda i: (0, 0)),
            pl.BlockSpec((1, D), lambda i: (0, 0)),
            pl.BlockSpec((_BM, 2), lambda i: (i, 0)),
            pl.BlockSpec(memory_space=pltpu.SMEM),
        ],
        out_specs=[pl.BlockSpec((8, _BM), lambda i: (0, i))],
        out_shape=[jax.ShapeDtypeStruct((8, N), jnp.float32)],
    )(p, b, wlin, blin, df, scal)[0]


def _edge_ex(el_v, er_v, src_ref, dst_ref, ch, j):
    sv = src_ref[ch, pl.ds(j * 16, 16)]
    dv = dst_ref[ch, pl.ds(j * 16, 16)]
    elv = plsc.load_gather(el_v, [sv])
    erv = plsc.load_gather(er_v, [dv])
    e = elv + erv
    e = jnp.where(e >= 0.0, e, e * jnp.float32(0.2))
    return dv, jnp.exp(e)


def _sc_body(feat_hbm, el_hbm, er_hbm, src_hbm, dst_hbm, zeros_hbm,
             out_hbm,
             el_v, er_v, own_src, own_dst, mir_src, mir_dst,
             alpha_v, rowbuf, denom_v, rowidx, out_sh, denom_sh, sem):
    c = lax.axis_index("c")
    s = lax.axis_index("s")

    pltpu.sync_copy(el_hbm, el_v)
    pltpu.sync_copy(er_hbm, er_v)
    pltpu.sync_copy(src_hbm.at[c, s], own_src)
    pltpu.sync_copy(dst_hbm.at[c, s], own_dst)
    pltpu.sync_copy(src_hbm.at[1 - c, s], mir_src)
    pltpu.sync_copy(dst_hbm.at[1 - c, s], mir_dst)

    pltpu.sync_copy(zeros_hbm.at[pl.ds(s * RPT, RPT)],
                    out_sh.at[pl.ds(s * RPT, RPT)])

    @pl.when(s == 0)
    def _():
        pltpu.sync_copy(zeros_hbm.at[pl.ds(0, DNR)], denom_sh)

    def _zero_row(r, _):
        zero = jnp.zeros((16,), jnp.float32)
        for j in range(8):
            denom_v[r, pl.ds(j * 16, 16)] = zero
        return 0

    lax.fori_loop(0, DNR, _zero_row, 0)

    for j in range(DNR // 16):
        rowidx[pl.ds(j * 16, 16)] = lax.iota(jnp.int32, 16) + j * 16

    def _p1(ch, _):
        for src_ref, dst_ref in ((own_src, own_dst), (mir_src, mir_dst)):
            for j in range(NG):
                dv, ex = _edge_ex(el_v, er_v, src_ref, dst_ref, ch, j)
                plsc.addupdate_scatter(
                    denom_v,
                    [jnp.right_shift(dv, 7), jnp.bitwise_and(dv, 127)],
                    ex)
        return 0

    lax.fori_loop(0, NCH, _p1, 0)

    pltpu.sync_copy(denom_v, denom_sh.at[rowidx], add=True)
    plsc.subcore_barrier()
    pltpu.sync_copy(denom_sh, denom_v)

    def _p2a(ch, _):
        for j in range(NG):
            dv, ex = _edge_ex(el_v, er_v, own_src, own_dst, ch, j)
            den = plsc.load_gather(
                denom_v,
                [jnp.right_shift(dv, 7), jnp.bitwise_and(dv, 127)])
            alpha_v[ch, pl.ds(j * 16, 16)] = ex / (den + jnp.float32(1e-16))
        return 0

    lax.fori_loop(0, NCH, _p2a, 0)

    def _p2b(ch, _):
        pltpu.async_copy(feat_hbm.at[own_src.at[ch]], rowbuf, sem).wait()

        def _scale(i, _):
            a = alpha_v[ch, i]
            for j in range(8):
                rowbuf[i, pl.ds(j * 16, 16)] = rowbuf[i, pl.ds(j * 16, 16)] * a
            return 0

        lax.fori_loop(0, CW, _scale, 0)
        pltpu.sync_copy(rowbuf, out_sh.at[own_dst.at[ch]], add=True)
        return 0

    lax.fori_loop(0, NCH, _p2b, 0)

    plsc.subcore_barrier()
    pltpu.sync_copy(out_sh.at[pl.ds(s * RPT, RPT)],
                    out_hbm.at[c, pl.ds(s * RPT, RPT)])


_sc_edges = pl.kernel(
    _sc_body,
    out_type=jax.ShapeDtypeStruct((NC, N, D), jnp.float32),
    mesh=plsc.VectorSubcoreMesh(core_axis_name="c", subcore_axis_name="s"),
    scratch_types=[
        pltpu.VMEM((N,), jnp.float32),
        pltpu.VMEM((N,), jnp.float32),
        pltpu.VMEM((NCH, CW), jnp.int32),
        pltpu.VMEM((NCH, CW), jnp.int32),
        pltpu.VMEM((NCH, CW), jnp.int32),
        pltpu.VMEM((NCH, CW), jnp.int32),
        pltpu.VMEM((NCH, CW), jnp.float32),
        pltpu.VMEM((CW, D), jnp.float32),
        pltpu.VMEM((DNR, D), jnp.float32),
        pltpu.VMEM((DNR,), jnp.int32),
        pltpu.VMEM_SHARED((N, D), jnp.float32),
        pltpu.VMEM_SHARED((DNR, D), jnp.float32),
        pltpu.SemaphoreType.DMA,
    ],
)


def kernel(h, edge_index1, edge_index2, dst_feat, e1, e2, cellID, clusters,
           alpha0, beta0, gamma0, dt,
           W1, attn_l1, attn_r1, b1, W2, attn_l2, attn_r2, b2, Wlin, blin):
    src1 = edge_index1[0].reshape(NC, NS, NCH, CW)
    dst1 = edge_index1[1].reshape(NC, NS, NCH, CW)
    src2 = edge_index2[0].reshape(NC, NS, NCH, CW)
    dst2 = edge_index2[1].reshape(NC, NS, NCH, CW)
    zeros = jnp.zeros((N, D), jnp.float32)

    feat1, elr1 = _tc_front(h, W1, attn_l1.reshape(1, D), attn_r1.reshape(1, D))
    p1 = _sc_edges(feat1, elr1[0], elr1[1], src1, dst1, zeros)
    feat2, elr2 = _tc_mid(p1, b1.reshape(1, D), W2,
                          attn_l2.reshape(1, D), attn_r2.reshape(1, D))
    p2 = _sc_edges(feat2, elr2[0], elr2[1], src2, dst2, zeros)

    wlin_pad = jnp.zeros((D, D), jnp.float32).at[:, :3].set(Wlin)
    blin_pad = jnp.zeros((1, D), jnp.float32).at[0, :3].set(blin)
    scal = jnp.stack([alpha0, beta0, gamma0, dt])
    res = _tc_head(p2, b2.reshape(1, D), wlin_pad, blin_pad, dst_feat, scal)

    us = dst_feat[:, 0]
    s = dst_feat[:, 1]
    return (res[0], res[1], res[2], res[3], res[4],
            e1, e2, us, s, cellID, clusters)


# speedup vs baseline: 14.6730x; 14.6730x over previous
import jax
import jax.numpy as jnp
from jax import lax
from jax.experimental import pallas as pl
from jax.experimental.pallas import tpu as pltpu
from jax.experimental.pallas import tpu_sc as plsc

N = 10000
D = 128
E = 320000
NC = 2
NS = 16
TPT = E // NC // NS
CW = 128
NCH = 80
EPT = NCH * CW
NG = CW // 16
SRPT = 624
SREM = N - NS * SRPT
DNR = 80

_BM = 1024


def _attn_tail(feat, al, ar, feat_ref, elr_ref):
    feat_ref[...] = feat
    el = jnp.sum(feat * al, axis=1)
    er = jnp.sum(feat * ar, axis=1)
    elr_ref[0:1, :] = el[None, :]
    elr_ref[1:2, :] = er[None, :]


def _tc_front_body(x_ref, w_ref, al_ref, ar_ref, feat_ref, elr_ref):
    feat = jnp.dot(x_ref[...], w_ref[...], preferred_element_type=jnp.float32)
    _attn_tail(feat, al_ref[...], ar_ref[...], feat_ref, elr_ref)


def _norm_x(u_ref, d_ref, b_ref):
    den = (d_ref[0:1, :] + d_ref[1:2, :]).T + jnp.float32(1e-16)
    return jax.nn.relu((u_ref[0] + u_ref[1]) / den + b_ref[...])


def _tc_mid_body(u_ref, d_ref, b_ref, w_ref, al_ref, ar_ref, feat_ref, elr_ref):
    x = _norm_x(u_ref, d_ref, b_ref)
    feat = jnp.dot(x, w_ref[...], preferred_element_type=jnp.float32)
    _attn_tail(feat, al_ref[...], ar_ref[...], feat_ref, elr_ref)


def _tc_head_body(u_ref, d_ref, b_ref, wlin_ref, blin_ref, df_ref, sc_ref,
                  res_ref):
    x = _norm_x(u_ref, d_ref, b_ref)
    z = jnp.dot(x, wlin_ref[...], preferred_element_type=jnp.float32)
    hout = jax.nn.sigmoid(z + blin_ref[...])
    hT = hout.T
    a0 = sc_ref[0]
    b0 = sc_ref[1]
    g0 = sc_ref[2]
    dtv = sc_ref[3]
    beta = hT[0:1, :] * b0
    gamma = hT[1:2, :] * g0
    alphas = hT[2:3, :] * a0
    dfT = df_ref[...].T
    us = dfT[0:1, :]
    s = dfT[1:2, :]
    res_ref[0:1, :] = us + (alphas - beta * us) * dtv
    res_ref[1:2, :] = s + (beta * us - gamma * s) * dtv
    res_ref[2:3, :] = alphas
    res_ref[3:4, :] = beta
    res_ref[4:5, :] = gamma
    res_ref[5:8, :] = jnp.zeros((3, res_ref.shape[1]), jnp.float32)


def _tc_front(x, w, al, ar):
    k = x.shape[1]
    grid = pl.cdiv(N, _BM)
    return pl.pallas_call(
        _tc_front_body,
        grid=(grid,),
        in_specs=[
            pl.BlockSpec((_BM, k), lambda i: (i, 0)),
            pl.BlockSpec((k, D), lambda i: (0, 0)),
            pl.BlockSpec((1, D), lambda i: (0, 0)),
            pl.BlockSpec((1, D), lambda i: (0, 0)),
        ],
        out_specs=[
            pl.BlockSpec((_BM, D), lambda i: (i, 0)),
            pl.BlockSpec((2, _BM), lambda i: (0, i)),
        ],
        out_shape=[
            jax.ShapeDtypeStruct((N, D), jnp.float32),
            jax.ShapeDtypeStruct((2, N), jnp.float32),
        ],
    )(x, w, al, ar)


def _tc_mid(u, d, b, w, al, ar):
    grid = pl.cdiv(N, _BM)
    return pl.pallas_call(
        _tc_mid_body,
        grid=(grid,),
        in_specs=[
            pl.BlockSpec((2, _BM, D), lambda i: (0, i, 0)),
            pl.BlockSpec((2, _BM), lambda i: (0, i)),
            pl.BlockSpec((1, D), lambda i: (0, 0)),
            pl.BlockSpec((D, D), lambda i: (0, 0)),
            pl.BlockSpec((1, D), lambda i: (0, 0)),
            pl.BlockSpec((1, D), lambda i: (0, 0)),
        ],
        out_specs=[
            pl.BlockSpec((_BM, D), lambda i: (i, 0)),
            pl.BlockSpec((2, _BM), lambda i: (0, i)),
        ],
        out_shape=[
            jax.ShapeDtypeStruct((N, D), jnp.float32),
            jax.ShapeDtypeStruct((2, N), jnp.float32),
        ],
    )(u, d, b, w, al, ar)


def _tc_head(u, d, b, wlin, blin, df, scal):
    grid = pl.cdiv(N, _BM)
    return pl.pallas_call(
        _tc_head_body,
        grid=(grid,),
        in_specs=[
            pl.BlockSpec((2, _BM, D), lambda i: (0, i, 0)),
            pl.BlockSpec((2, _BM), lambda i: (0, i)),
            pl.BlockSpec((1, D), lambda i: (0, 0)),
            pl.BlockSpec((D, D), lambda i: (0, 0)),
            pl.BlockSpec((1, D), lambda i: (0, 0)),
            pl.BlockSpec((_BM, 2), lambda i: (i, 0)),
            pl.BlockSpec(memory_space=pltpu.SMEM),
        ],
        out_specs=[pl.BlockSpec((8, _BM), lambda i: (0, i))],
        out_shape=[jax.ShapeDtypeStruct((8, N), jnp.float32)],
    )(u, d, b, wlin, blin, df, scal)[0]


def _sc_scalars_body(el_hbm, er_hbm, src_hbm, dst_hbm, zeros_hbm,
                     ex_hbm, den_hbm,
                     el_v, er_v, own_src, own_dst, ex_own, denom_v, rowidx,
                     denom_sh):
    c = lax.axis_index("c")
    s = lax.axis_index("s")

    pltpu.sync_copy(el_hbm, el_v)
    pltpu.sync_copy(er_hbm, er_v)
    pltpu.sync_copy(src_hbm.at[c, s], own_src)
    pltpu.sync_copy(dst_hbm.at[c, s], own_dst)

    @pl.when(s == 0)
    def _():
        pltpu.sync_copy(zeros_hbm.at[pl.ds(0, DNR)], denom_sh)

    def _zero_row(r, _):
        zero = jnp.zeros((16,), jnp.float32)
        for j in range(8):
            denom_v[r, pl.ds(j * 16, 16)] = zero
        return 0

    lax.fori_loop(0, DNR, _zero_row, 0)

    for j in range(DNR // 16):
        rowidx[pl.ds(j * 16, 16)] = lax.iota(jnp.int32, 16) + j * 16

    plsc.subcore_barrier()

    def _p1(ch, _):
        base = ch * CW
        for j in range(NG):
            sv = own_src[ch, pl.ds(j * 16, 16)]
            dv = own_dst[ch, pl.ds(j * 16, 16)]
            e = plsc.load_gather(el_v, [sv]) + plsc.load_gather(er_v, [dv])
            e = jnp.where(e >= 0.0, e, e * jnp.float32(0.2))
            ex = jnp.exp(e)
            flat = lax.iota(jnp.int32, 16) + (base + j * 16)
            ex = jnp.where(flat < TPT, ex, jnp.float32(0.0))
            ex_own[ch, pl.ds(j * 16, 16)] = ex
            plsc.addupdate_scatter(
                denom_v,
                [jnp.right_shift(dv, 7), jnp.bitwise_and(dv, 127)],
                ex)
        return 0

    lax.fori_loop(0, NCH, _p1, 0)

    pltpu.sync_copy(ex_own, ex_hbm.at[c, s])
    pltpu.sync_copy(denom_v, denom_sh.at[rowidx], add=True)
    plsc.subcore_barrier()

    @pl.when(s == 0)
    def _():
        pltpu.sync_copy(denom_sh, den_hbm.at[c])


_sc_scalars = pl.kernel(
    _sc_scalars_body,
    out_type=[
        jax.ShapeDtypeStruct((NC, NS, NCH, CW), jnp.float32),
        jax.ShapeDtypeStruct((NC, DNR, D), jnp.float32),
    ],
    mesh=plsc.VectorSubcoreMesh(core_axis_name="c", subcore_axis_name="s",
                                num_cores=NC, num_subcores=NS),
    scratch_types=[
        pltpu.VMEM((N,), jnp.float32),
        pltpu.VMEM((N,), jnp.float32),
        pltpu.VMEM((NCH, CW), jnp.int32),
        pltpu.VMEM((NCH, CW), jnp.int32),
        pltpu.VMEM((NCH, CW), jnp.float32),
        pltpu.VMEM((DNR, D), jnp.float32),
        pltpu.VMEM((DNR,), jnp.int32),
        pltpu.VMEM_SHARED((DNR, D), jnp.float32),
    ],
    compiler_params=pltpu.CompilerParams(needs_layout_passes=False),
)


def _sc_agg_body(feat_hbm, src_hbm, dst_hbm, exv_hbm, zeros_hbm,
                 out_hbm,
                 own_src, own_dst, ex_own, rowbuf, out_sh):
    c = lax.axis_index("c")
    s = lax.axis_index("s")

    pltpu.sync_copy(src_hbm.at[c, s], own_src)
    pltpu.sync_copy(dst_hbm.at[c, s], own_dst)
    pltpu.sync_copy(exv_hbm.at[c, s], ex_own)

    pltpu.sync_copy(zeros_hbm.at[pl.ds(s * SRPT, SRPT)],
                    out_sh.at[pl.ds(s * SRPT, SRPT)])

    @pl.when(s == NS - 1)
    def _():
        pltpu.sync_copy(zeros_hbm.at[pl.ds(NS * SRPT, SREM)],
                        out_sh.at[pl.ds(NS * SRPT, SREM)])

    plsc.subcore_barrier()

    def _p2(ch, _):
        pltpu.sync_copy(feat_hbm.at[own_src.at[ch]], rowbuf)

        def _scale(i, _):
            chv = jnp.zeros((16,), jnp.int32) + ch
            iv = jnp.zeros((16,), jnp.int32) + i
            a = plsc.load_gather(ex_own, [chv, iv])
            for j in range(8):
                rowbuf[i, pl.ds(j * 16, 16)] = rowbuf[i, pl.ds(j * 16, 16)] * a
            return 0

        lax.fori_loop(0, CW, _scale, 0)
        pltpu.sync_copy(rowbuf, out_sh.at[own_dst.at[ch]], add=True)
        return 0

    lax.fori_loop(0, NCH, _p2, 0)

    plsc.subcore_barrier()
    pltpu.sync_copy(out_sh.at[pl.ds(s * SRPT, SRPT)],
                    out_hbm.at[c, pl.ds(s * SRPT, SRPT)])

    @pl.when(s == NS - 1)
    def _():
        pltpu.sync_copy(out_sh.at[pl.ds(NS * SRPT, SREM)],
                        out_hbm.at[c, pl.ds(NS * SRPT, SREM)])


_sc_agg = pl.kernel(
    _sc_agg_body,
    out_type=jax.ShapeDtypeStruct((NC, N, D), jnp.float32),
    mesh=plsc.VectorSubcoreMesh(core_axis_name="c", subcore_axis_name="s",
                                num_cores=NC, num_subcores=NS),
    scratch_types=[
        pltpu.VMEM((NCH, CW), jnp.int32),
        pltpu.VMEM((NCH, CW), jnp.int32),
        pltpu.VMEM((NCH, CW), jnp.float32),
        pltpu.VMEM((CW, D), jnp.float32),
        pltpu.VMEM_SHARED((N, D), jnp.float32),
    ],
    compiler_params=pltpu.CompilerParams(needs_layout_passes=False),
)


def _pad_edges(row):
    per_tile = row.reshape(NC, NS, TPT)
    pad = jnp.zeros((NC, NS, EPT - TPT), jnp.int32)
    return jnp.concatenate([per_tile, pad], axis=-1).reshape(NC, NS, NCH, CW)


def _gat_layer(feat, elr, src4, dst4, zeros):
    ex, den = _sc_scalars(elr[0], elr[1], src4, dst4, zeros)
    u = _sc_agg(feat, src4, dst4, ex, zeros)
    d2 = den.reshape(NC, DNR * D)[:, :N]
    return u, d2


def kernel(h, edge_index1, edge_index2, dst_feat, e1, e2, cellID, clusters,
           alpha0, beta0, gamma0, dt,
           W1, attn_l1, attn_r1, b1, W2, attn_l2, attn_r2, b2, Wlin, blin):
    src1 = _pad_edges(edge_index1[0])
    dst1 = _pad_edges(edge_index1[1])
    src2 = _pad_edges(edge_index2[0])
    dst2 = _pad_edges(edge_index2[1])
    zeros = jnp.zeros((N, D), jnp.float32)

    feat1, elr1 = _tc_front(h, W1, attn_l1.reshape(1, D), attn_r1.reshape(1, D))
    u1, d1 = _gat_layer(feat1, elr1, src1, dst1, zeros)
    feat2, elr2 = _tc_mid(u1, d1, b1.reshape(1, D), W2,
                          attn_l2.reshape(1, D), attn_r2.reshape(1, D))
    u2, d2 = _gat_layer(feat2, elr2, src2, dst2, zeros)

    wlin_pad = jnp.zeros((D, D), jnp.float32).at[:, :3].set(Wlin)
    blin_pad = jnp.zeros((1, D), jnp.float32).at[0, :3].set(blin)
    scal = jnp.stack([alpha0, beta0, gamma0, dt])
    res = _tc_head(u2, d2, b2.reshape(1, D), wlin_pad, blin_pad,
                   dst_feat, scal)

    us = dst_feat[:, 0]
    s = dst_feat[:, 1]
    return (res[0], res[1], res[2], res[3], res[4],
            e1, e2, us, s, cellID, clusters)


# speedup vs baseline: 17.6580x; 1.2034x over previous
import jax
import jax.numpy as jnp
from jax import lax
from jax.experimental import pallas as pl
from jax.experimental.pallas import tpu as pltpu
from jax.experimental.pallas import tpu_sc as plsc

N = 10000
D = 128
E = 320000
NC = 2
NS = 16
TPT = E // NC // NS
CW = 128
NCH = 80
EPT = NCH * CW
NG = CW // 16
SRPT = 624
SREM = N - NS * SRPT
DNR = 80
PPC = 8

_BM = 1024


def _attn_tail(feat, al, ar, feat_ref, elr_ref):
    feat_ref[...] = feat
    el = jnp.sum(feat * al, axis=1)
    er = jnp.sum(feat * ar, axis=1)
    elr_ref[0:1, :] = el[None, :]
    elr_ref[1:2, :] = er[None, :]


def _tc_front_body(x_ref, w_ref, al_ref, ar_ref, feat_ref, elr_ref):
    feat = jnp.dot(x_ref[...], w_ref[...], preferred_element_type=jnp.float32)
    _attn_tail(feat, al_ref[...], ar_ref[...], feat_ref, elr_ref)


def _norm_x(u_ref, d_ref, b_ref):
    den = (d_ref[0:1, :] + d_ref[1:2, :]).T + jnp.float32(1e-16)
    return jax.nn.relu((u_ref[0] + u_ref[1]) / den + b_ref[...])


def _tc_mid_body(u_ref, d_ref, b_ref, w_ref, al_ref, ar_ref, feat_ref, elr_ref):
    x = _norm_x(u_ref, d_ref, b_ref)
    feat = jnp.dot(x, w_ref[...], preferred_element_type=jnp.float32)
    _attn_tail(feat, al_ref[...], ar_ref[...], feat_ref, elr_ref)


def _tc_head_body(u_ref, d_ref, b_ref, wlin_ref, blin_ref, df_ref, sc_ref,
                  res_ref):
    x = _norm_x(u_ref, d_ref, b_ref)
    z = jnp.dot(x, wlin_ref[...], preferred_element_type=jnp.float32)
    hout = jax.nn.sigmoid(z + blin_ref[...])
    hT = hout.T
    a0 = sc_ref[0]
    b0 = sc_ref[1]
    g0 = sc_ref[2]
    dtv = sc_ref[3]
    beta = hT[0:1, :] * b0
    gamma = hT[1:2, :] * g0
    alphas = hT[2:3, :] * a0
    dfT = df_ref[...].T
    us = dfT[0:1, :]
    s = dfT[1:2, :]
    res_ref[0:1, :] = us + (alphas - beta * us) * dtv
    res_ref[1:2, :] = s + (beta * us - gamma * s) * dtv
    res_ref[2:3, :] = alphas
    res_ref[3:4, :] = beta
    res_ref[4:5, :] = gamma
    res_ref[5:8, :] = jnp.zeros((3, res_ref.shape[1]), jnp.float32)


def _tc_front(x, w, al, ar):
    k = x.shape[1]
    grid = pl.cdiv(N, _BM)
    return pl.pallas_call(
        _tc_front_body,
        grid=(grid,),
        in_specs=[
            pl.BlockSpec((_BM, k), lambda i: (i, 0)),
            pl.BlockSpec((k, D), lambda i: (0, 0)),
            pl.BlockSpec((1, D), lambda i: (0, 0)),
            pl.BlockSpec((1, D), lambda i: (0, 0)),
        ],
        out_specs=[
            pl.BlockSpec((_BM, D), lambda i: (i, 0)),
            pl.BlockSpec((2, _BM), lambda i: (0, i)),
        ],
        out_shape=[
            jax.ShapeDtypeStruct((N, D), jnp.float32),
            jax.ShapeDtypeStruct((2, N), jnp.float32),
        ],
    )(x, w, al, ar)


def _tc_mid(u, d, b, w, al, ar):
    grid = pl.cdiv(N, _BM)
    return pl.pallas_call(
        _tc_mid_body,
        grid=(grid,),
        in_specs=[
            pl.BlockSpec((2, _BM, D), lambda i: (0, i, 0)),
            pl.BlockSpec((2, _BM), lambda i: (0, i)),
            pl.BlockSpec((1, D), lambda i: (0, 0)),
            pl.BlockSpec((D, D), lambda i: (0, 0)),
            pl.BlockSpec((1, D), lambda i: (0, 0)),
            pl.BlockSpec((1, D), lambda i: (0, 0)),
        ],
        out_specs=[
            pl.BlockSpec((_BM, D), lambda i: (i, 0)),
            pl.BlockSpec((2, _BM), lambda i: (0, i)),
        ],
        out_shape=[
            jax.ShapeDtypeStruct((N, D), jnp.float32),
            jax.ShapeDtypeStruct((2, N), jnp.float32),
        ],
    )(u, d, b, w, al, ar)


def _tc_head(u, d, b, wlin, blin, df, scal):
    grid = pl.cdiv(N, _BM)
    return pl.pallas_call(
        _tc_head_body,
        grid=(grid,),
        in_specs=[
            pl.BlockSpec((2, _BM, D), lambda i: (0, i, 0)),
            pl.BlockSpec((2, _BM), lambda i: (0, i)),
            pl.BlockSpec((1, D), lambda i: (0, 0)),
            pl.BlockSpec((D, D), lambda i: (0, 0)),
            pl.BlockSpec((1, D), lambda i: (0, 0)),
            pl.BlockSpec((_BM, 2), lambda i: (i, 0)),
            pl.BlockSpec(memory_space=pltpu.SMEM),
        ],
        out_specs=[pl.BlockSpec((8, _BM), lambda i: (0, i))],
        out_shape=[jax.ShapeDtypeStruct((8, N), jnp.float32)],
    )(u, d, b, wlin, blin, df, scal)[0]


def _sc_scalars_body(el_hbm, er_hbm, src_hbm, dst_hbm, zeros_hbm,
                     ex_hbm, den_hbm,
                     el_v, er_v, own_src, own_dst, ex_own, denom_v, rowidx,
                     denom_sh):
    c = lax.axis_index("c")
    s = lax.axis_index("s")

    pltpu.sync_copy(el_hbm, el_v)
    pltpu.sync_copy(er_hbm, er_v)
    pltpu.sync_copy(src_hbm.at[c, s], own_src)
    pltpu.sync_copy(dst_hbm.at[c, s], own_dst)

    @pl.when(s == 0)
    def _():
        pltpu.sync_copy(zeros_hbm.at[pl.ds(0, DNR)], denom_sh)

    def _zero_row(r, _):
        zero = jnp.zeros((16,), jnp.float32)
        for j in range(8):
            denom_v[r, pl.ds(j * 16, 16)] = zero
        return 0

    lax.fori_loop(0, DNR, _zero_row, 0)

    for j in range(DNR // 16):
        rowidx[pl.ds(j * 16, 16)] = lax.iota(jnp.int32, 16) + j * 16

    plsc.subcore_barrier()

    def _p1(ch, _):
        base = ch * CW
        for j in range(NG):
            sv = own_src[ch, pl.ds(j * 16, 16)]
            dv = own_dst[ch, pl.ds(j * 16, 16)]
            e = plsc.load_gather(el_v, [sv]) + plsc.load_gather(er_v, [dv])
            e = jnp.where(e >= 0.0, e, e * jnp.float32(0.2))
            ex = jnp.exp(e)
            flat = lax.iota(jnp.int32, 16) + (base + j * 16)
            ex = jnp.where(flat < TPT, ex, jnp.float32(0.0))
            ex_own[ch, pl.ds(j * 16, 16)] = ex
            plsc.addupdate_scatter(
                denom_v,
                [jnp.right_shift(dv, 7), jnp.bitwise_and(dv, 127)],
                ex)
        return 0

    lax.fori_loop(0, NCH, _p1, 0)

    pltpu.sync_copy(ex_own, ex_hbm.at[c, s])
    pltpu.sync_copy(denom_v, denom_sh.at[rowidx], add=True)
    plsc.subcore_barrier()

    @pl.when(s == 0)
    def _():
        pltpu.sync_copy(denom_sh, den_hbm.at[c])


_sc_scalars = pl.kernel(
    _sc_scalars_body,
    out_type=[
        jax.ShapeDtypeStruct((NC, NS, NCH, CW), jnp.float32),
        jax.ShapeDtypeStruct((NC, DNR, D), jnp.float32),
    ],
    mesh=plsc.VectorSubcoreMesh(core_axis_name="c", subcore_axis_name="s",
                                num_cores=NC, num_subcores=NS),
    scratch_types=[
        pltpu.VMEM((N,), jnp.float32),
        pltpu.VMEM((N,), jnp.float32),
        pltpu.VMEM((NCH, CW), jnp.int32),
        pltpu.VMEM((NCH, CW), jnp.int32),
        pltpu.VMEM((NCH, CW), jnp.float32),
        pltpu.VMEM((DNR, D), jnp.float32),
        pltpu.VMEM((DNR,), jnp.int32),
        pltpu.VMEM_SHARED((DNR, D), jnp.float32),
    ],
    compiler_params=pltpu.CompilerParams(needs_layout_passes=False),
)


def _sc_agg_body(feat_hbm, src_hbm, dst_hbm, exv_hbm, zeros_hbm,
                 out_hbm,
                 src_pc, dst_pc, ex_own, rb0, rb1, out_sh, gs0, gs1):
    c = lax.axis_index("c")
    s = lax.axis_index("s")

    pltpu.sync_copy(exv_hbm.at[c, s], ex_own)

    pltpu.sync_copy(zeros_hbm.at[pl.ds(s * SRPT, SRPT)],
                    out_sh.at[pl.ds(s * SRPT, SRPT)])

    @pl.when(s == NS - 1)
    def _():
        pltpu.sync_copy(zeros_hbm.at[pl.ds(NS * SRPT, SREM)],
                        out_sh.at[pl.ds(NS * SRPT, SREM)])

    plsc.subcore_barrier()

    rbs = (rb0, rb1)
    gss = (gs0, gs1)

    def _scale(ch, rb):
        def _grp(g, _):
            av = ex_own[ch, pl.ds(g * 16, 16)]
            for l in range(16):
                a = av[l]
                r = g * 16 + l
                for j in range(8):
                    rb[r, pl.ds(j * 16, 16)] = rb[r, pl.ds(j * 16, 16)] * a
            return 0

        lax.fori_loop(0, CW // 16, _grp, 0)

    def _piece(p, _):
        pltpu.sync_copy(src_hbm.at[c, s, pl.ds(p * PPC, PPC)], src_pc)
        pltpu.sync_copy(dst_hbm.at[c, s, pl.ds(p * PPC, PPC)], dst_pc)
        pltpu.async_copy(feat_hbm.at[src_pc.at[0]], rb0, gs0)
        for jj in range(PPC):
            b = jj % 2
            if jj + 1 < PPC:
                pltpu.async_copy(feat_hbm.at[src_pc.at[jj + 1]],
                                 rbs[1 - b], gss[1 - b])
            pltpu.make_async_copy(feat_hbm.at[src_pc.at[jj]],
                                  rbs[b], gss[b]).wait()
            _scale(p * PPC + jj, rbs[b])
            pltpu.sync_copy(rbs[b], out_sh.at[dst_pc.at[jj]], add=True)
        return 0

    lax.fori_loop(0, NCH // PPC, _piece, 0)

    plsc.subcore_barrier()
    pltpu.sync_copy(out_sh.at[pl.ds(s * SRPT, SRPT)],
                    out_hbm.at[c, pl.ds(s * SRPT, SRPT)])

    @pl.when(s == NS - 1)
    def _():
        pltpu.sync_copy(out_sh.at[pl.ds(NS * SRPT, SREM)],
                        out_hbm.at[c, pl.ds(NS * SRPT, SREM)])


_sc_agg = pl.kernel(
    _sc_agg_body,
    out_type=jax.ShapeDtypeStruct((NC, N, D), jnp.float32),
    mesh=plsc.VectorSubcoreMesh(core_axis_name="c", subcore_axis_name="s",
                                num_cores=NC, num_subcores=NS),
    scratch_types=[
        pltpu.VMEM((PPC, CW), jnp.int32),
        pltpu.VMEM((PPC, CW), jnp.int32),
        pltpu.VMEM((NCH, CW), jnp.float32),
        pltpu.VMEM((CW, D), jnp.float32),
        pltpu.VMEM((CW, D), jnp.float32),
        pltpu.VMEM_SHARED((N, D), jnp.float32),
        pltpu.SemaphoreType.DMA,
        pltpu.SemaphoreType.DMA,
    ],
    compiler_params=pltpu.CompilerParams(needs_layout_passes=False),
)


def _pad_edges(row):
    per_tile = row.reshape(NC, NS, TPT)
    pad = jnp.zeros((NC, NS, EPT - TPT), jnp.int32)
    return jnp.concatenate([per_tile, pad], axis=-1).reshape(NC, NS, NCH, CW)


def _gat_layer(feat, elr, src4, dst4, zeros):
    ex, den = _sc_scalars(elr[0], elr[1], src4, dst4, zeros)
    u = _sc_agg(feat, src4, dst4, ex, zeros)
    d2 = den.reshape(NC, DNR * D)[:, :N]
    return u, d2


def kernel(h, edge_index1, edge_index2, dst_feat, e1, e2, cellID, clusters,
           alpha0, beta0, gamma0, dt,
           W1, attn_l1, attn_r1, b1, W2, attn_l2, attn_r2, b2, Wlin, blin):
    src1 = _pad_edges(edge_index1[0])
    dst1 = _pad_edges(edge_index1[1])
    src2 = _pad_edges(edge_index2[0])
    dst2 = _pad_edges(edge_index2[1])
    zeros = jnp.zeros((N, D), jnp.float32)

    feat1, elr1 = _tc_front(h, W1, attn_l1.reshape(1, D), attn_r1.reshape(1, D))
    u1, d1 = _gat_layer(feat1, elr1, src1, dst1, zeros)
    feat2, elr2 = _tc_mid(u1, d1, b1.reshape(1, D), W2,
                          attn_l2.reshape(1, D), attn_r2.reshape(1, D))
    u2, d2 = _gat_layer(feat2, elr2, src2, dst2, zeros)

    wlin_pad = jnp.zeros((D, D), jnp.float32).at[:, :3].set(Wlin)
    blin_pad = jnp.zeros((1, D), jnp.float32).at[0, :3].set(blin)
    scal = jnp.stack([alpha0, beta0, gamma0, dt])
    res = _tc_head(u2, d2, b2.reshape(1, D), wlin_pad, blin_pad,
                   dst_feat, scal)

    us = dst_feat[:, 0]
    s = dst_feat[:, 1]
    return (res[0], res[1], res[2], res[3], res[4],
            e1, e2, us, s, cellID, clusters)


# speedup vs baseline: 17.7093x; 1.0029x over previous
import numpy as _np

import jax
import jax.numpy as jnp
from jax import lax
from jax.experimental import pallas as pl
from jax.experimental.pallas import tpu as pltpu
from jax.experimental.pallas import tpu_sc as plsc

N = 10000
D = 128
E = 320000
NC = 2
NS = 16
TPT = E // NC // NS
CW = 128
NCH = 80
EPT = NCH * CW
NG = CW // 16
SRPT = 624
SREM = N - NS * SRPT
DNR = 80
PPC = 8

_BM = 1024


def _attn_tail(feat, al, ar, feat_ref, elr_ref):
    feat_ref[...] = feat
    el = jnp.sum(feat * al, axis=1)
    er = jnp.sum(feat * ar, axis=1)
    elr_ref[0:1, :] = el[None, :]
    elr_ref[1:2, :] = er[None, :]


def _tc_front_body(x_ref, w_ref, al_ref, ar_ref, feat_ref, elr_ref):
    feat = jnp.dot(x_ref[...], w_ref[...], preferred_element_type=jnp.float32)
    _attn_tail(feat, al_ref[...], ar_ref[...], feat_ref, elr_ref)


def _norm_x(u_ref, d_ref, b_ref):
    den = (d_ref[0:1, :] + d_ref[1:2, :]).T + jnp.float32(1e-16)
    return jax.nn.relu((u_ref[0] + u_ref[1]) / den + b_ref[...])


def _tc_mid_body(u_ref, d_ref, b_ref, w_ref, al_ref, ar_ref, feat_ref,
                 elr_ref):
    x = _norm_x(u_ref, d_ref, b_ref)
    feat = jnp.dot(x, w_ref[...], preferred_element_type=jnp.float32)
    _attn_tail(feat, al_ref[...], ar_ref[...], feat_ref, elr_ref)


def _tc_head_body(u_ref, d_ref, b_ref, wlin_ref, blin_ref, df_ref, sc_ref,
                  res_ref):
    x = _norm_x(u_ref, d_ref, b_ref)
    z = jnp.dot(x, wlin_ref[...], preferred_element_type=jnp.float32)
    hout = jax.nn.sigmoid(z + blin_ref[...])
    hT = hout.T
    a0 = sc_ref[0]
    b0 = sc_ref[1]
    g0 = sc_ref[2]
    dtv = sc_ref[3]
    beta = hT[0:1, :] * b0
    gamma = hT[1:2, :] * g0
    alphas = hT[2:3, :] * a0
    dfT = df_ref[...].T
    us = dfT[0:1, :]
    s = dfT[1:2, :]
    res_ref[0:1, :] = us + (alphas - beta * us) * dtv
    res_ref[1:2, :] = s + (beta * us - gamma * s) * dtv
    res_ref[2:3, :] = alphas
    res_ref[3:4, :] = beta
    res_ref[4:5, :] = gamma
    res_ref[5:8, :] = jnp.zeros((3, res_ref.shape[1]), jnp.float32)


def _tc_front(x, w, al, ar):
    k = x.shape[1]
    grid = pl.cdiv(N, _BM)
    return pl.pallas_call(
        _tc_front_body,
        grid=(grid,),
        in_specs=[
            pl.BlockSpec((_BM, k), lambda i: (i, 0)),
            pl.BlockSpec((k, D), lambda i: (0, 0)),
            pl.BlockSpec((1, D), lambda i: (0, 0)),
            pl.BlockSpec((1, D), lambda i: (0, 0)),
        ],
        out_specs=[
            pl.BlockSpec((_BM, D), lambda i: (i, 0)),
            pl.BlockSpec((2, _BM), lambda i: (0, i)),
        ],
        out_shape=[
            jax.ShapeDtypeStruct((N, D), jnp.float32),
            jax.ShapeDtypeStruct((2, N), jnp.float32),
        ],
    )(x, w, al, ar)


def _tc_mid(u, d, b, w, al, ar):
    grid = pl.cdiv(N, _BM)
    return pl.pallas_call(
        _tc_mid_body,
        grid=(grid,),
        in_specs=[
            pl.BlockSpec((2, _BM, D), lambda i: (0, i, 0)),
            pl.BlockSpec((2, _BM), lambda i: (0, i)),
            pl.BlockSpec((1, D), lambda i: (0, 0)),
            pl.BlockSpec((D, D), lambda i: (0, 0)),
            pl.BlockSpec((1, D), lambda i: (0, 0)),
            pl.BlockSpec((1, D), lambda i: (0, 0)),
        ],
        out_specs=[
            pl.BlockSpec((_BM, D), lambda i: (i, 0)),
            pl.BlockSpec((2, _BM), lambda i: (0, i)),
        ],
        out_shape=[
            jax.ShapeDtypeStruct((N, D), jnp.float32),
            jax.ShapeDtypeStruct((2, N), jnp.float32),
        ],
    )(u, d, b, w, al, ar)


def _tc_head(u, d, b, wlin, blin, df, scal):
    grid = pl.cdiv(N, _BM)
    return pl.pallas_call(
        _tc_head_body,
        grid=(grid,),
        in_specs=[
            pl.BlockSpec((2, _BM, D), lambda i: (0, i, 0)),
            pl.BlockSpec((2, _BM), lambda i: (0, i)),
            pl.BlockSpec((1, D), lambda i: (0, 0)),
            pl.BlockSpec((D, D), lambda i: (0, 0)),
            pl.BlockSpec((1, D), lambda i: (0, 0)),
            pl.BlockSpec((_BM, 2), lambda i: (i, 0)),
            pl.BlockSpec(memory_space=pltpu.SMEM),
        ],
        out_specs=[pl.BlockSpec((8, _BM), lambda i: (0, i))],
        out_shape=[jax.ShapeDtypeStruct((8, N), jnp.float32)],
    )(u, d, b, wlin, blin, df, scal)[0]


def _sc_scalars_body(el_hbm, er_hbm, src_hbm, dst_hbm, zeros_hbm,
                     ex_hbm, den_hbm,
                     el_v, er_v, own_src, own_dst, ex_own, denom_v, rowidx,
                     denom_sh):
    c = lax.axis_index("c")
    s = lax.axis_index("s")

    pltpu.sync_copy(el_hbm, el_v)
    pltpu.sync_copy(er_hbm, er_v)
    pltpu.sync_copy(src_hbm.at[c, s], own_src)
    pltpu.sync_copy(dst_hbm.at[c, s], own_dst)

    @pl.when(s == 0)
    def _():
        pltpu.sync_copy(zeros_hbm.at[pl.ds(0, DNR)], denom_sh)

    def _zero_row(r, _):
        zero = jnp.zeros((16,), jnp.float32)
        for j in range(8):
            denom_v[r, pl.ds(j * 16, 16)] = zero
        return 0

    lax.fori_loop(0, DNR, _zero_row, 0)

    for j in range(DNR // 16):
        rowidx[pl.ds(j * 16, 16)] = lax.iota(jnp.int32, 16) + j * 16

    plsc.subcore_barrier()

    def _p1(ch, _):
        base = ch * CW
        for j in range(NG):
            sv = own_src[ch, pl.ds(j * 16, 16)]
            dv = own_dst[ch, pl.ds(j * 16, 16)]
            e = plsc.load_gather(el_v, [sv]) + plsc.load_gather(er_v, [dv])
            e = jnp.where(e >= 0.0, e, e * jnp.float32(0.2))
            ex = jnp.exp(e)
            flat = lax.iota(jnp.int32, 16) + (base + j * 16)
            ex = jnp.where(flat < TPT, ex, jnp.float32(0.0))
            ex_own[ch, pl.ds(j * 16, 16)] = ex
            plsc.addupdate_scatter(
                denom_v,
                [jnp.right_shift(dv, 7), jnp.bitwise_and(dv, 127)],
                ex)
        return 0

    lax.fori_loop(0, NCH, _p1, 0)

    pltpu.sync_copy(ex_own, ex_hbm.at[c, s])
    pltpu.sync_copy(denom_v, denom_sh.at[rowidx], add=True)
    plsc.subcore_barrier()

    @pl.when(s == 0)
    def _():
        pltpu.sync_copy(denom_sh, den_hbm.at[c])


_sc_scalars = pl.kernel(
    _sc_scalars_body,
    out_type=[
        jax.ShapeDtypeStruct((NC, NS, NCH, CW), jnp.float32),
        jax.ShapeDtypeStruct((NC, DNR, D), jnp.float32),
    ],
    mesh=plsc.VectorSubcoreMesh(core_axis_name="c", subcore_axis_name="s",
                                num_cores=NC, num_subcores=NS),
    scratch_types=[
        pltpu.VMEM((N,), jnp.float32),
        pltpu.VMEM((N,), jnp.float32),
        pltpu.VMEM((NCH, CW), jnp.int32),
        pltpu.VMEM((NCH, CW), jnp.int32),
        pltpu.VMEM((NCH, CW), jnp.float32),
        pltpu.VMEM((DNR, D), jnp.float32),
        pltpu.VMEM((DNR,), jnp.int32),
        pltpu.VMEM_SHARED((DNR, D), jnp.float32),
    ],
    compiler_params=pltpu.CompilerParams(needs_layout_passes=False),
)


def _sc_agg_body(feat_hbm, srcf_hbm, dst_hbm, exv_hbm, zeros_hbm,
                 out_hbm,
                 src_pc, dst_pc, ex_own, rb0, rb1, out_sh, gs0, gs1):
    c = lax.axis_index("c")
    s = lax.axis_index("s")

    pltpu.sync_copy(exv_hbm.at[c, s], ex_own)

    pltpu.sync_copy(zeros_hbm.at[pl.ds(s * SRPT, SRPT)],
                    out_sh.at[pl.ds(s * SRPT, SRPT)])

    @pl.when(s == NS - 1)
    def _():
        pltpu.sync_copy(zeros_hbm.at[pl.ds(NS * SRPT, SREM)],
                        out_sh.at[pl.ds(NS * SRPT, SREM)])

    plsc.subcore_barrier()

    rbs = (rb0, rb1)
    gss = (gs0, gs1)

    def _scale(ch, rb):
        def _grp(g, _):
            av = ex_own[ch, pl.ds(g * 16, 16)]
            for l in range(16):
                a = av[l]
                r = g * 16 + l
                for j in range(8):
                    rb[r, pl.ds(j * 16, 16)] = rb[r, pl.ds(j * 16, 16)] * a
            return 0

        lax.fori_loop(0, CW // 16, _grp, 0)

    def _piece(p, _):
        pltpu.sync_copy(srcf_hbm.at[c, s, pl.ds(p * PPC * CW, PPC * CW)],
                        src_pc)
        pltpu.sync_copy(dst_hbm.at[c, s, pl.ds(p * PPC, PPC)], dst_pc)
        pltpu.async_copy(feat_hbm.at[src_pc.at[pl.ds(0, CW)]], rb0, gs0)
        for jj in range(PPC):
            b = jj % 2
            if jj + 1 < PPC:
                pltpu.async_copy(
                    feat_hbm.at[src_pc.at[pl.ds((jj + 1) * CW, CW)]],
                    rbs[1 - b], gss[1 - b])
            pltpu.make_async_copy(feat_hbm.at[src_pc.at[pl.ds(jj * CW, CW)]],
                                  rbs[b], gss[b]).wait()
            _scale(p * PPC + jj, rbs[b])
            pltpu.sync_copy(rbs[b], out_sh.at[dst_pc.at[jj]], add=True)
        return 0

    lax.fori_loop(0, NCH // PPC, _piece, 0)

    plsc.subcore_barrier()
    pltpu.sync_copy(out_sh.at[pl.ds(s * SRPT, SRPT)],
                    out_hbm.at[c, pl.ds(s * SRPT, SRPT)])

    @pl.when(s == NS - 1)
    def _():
        pltpu.sync_copy(out_sh.at[pl.ds(NS * SRPT, SREM)],
                        out_hbm.at[c, pl.ds(NS * SRPT, SREM)])


_sc_agg = pl.kernel(
    _sc_agg_body,
    out_type=jax.ShapeDtypeStruct((NC, N, D), jnp.float32),
    mesh=plsc.VectorSubcoreMesh(core_axis_name="c", subcore_axis_name="s",
                                num_cores=NC, num_subcores=NS),
    scratch_types=[
        pltpu.VMEM((PPC * CW,), jnp.int32),
        pltpu.VMEM((PPC, CW), jnp.int32),
        pltpu.VMEM((NCH, CW), jnp.float32),
        pltpu.VMEM((CW, D), jnp.float32),
        pltpu.VMEM((CW, D), jnp.float32),
        pltpu.VMEM_SHARED((N, D), jnp.float32),
        pltpu.SemaphoreType.DMA,
        pltpu.SemaphoreType.DMA,
    ],
    compiler_params=pltpu.CompilerParams(needs_layout_passes=False),
)


def _pad_edges(row):
    per_tile = row.reshape(NC, NS, TPT)
    pad = jnp.zeros((NC, NS, EPT - TPT), jnp.int32)
    return jnp.concatenate([per_tile, pad], axis=-1).reshape(NC, NS, NCH, CW)


def _gat_layer(feat, elr, src4, dst4, zeros):
    ex, den = _sc_scalars(elr[0], elr[1], src4, dst4, zeros)
    srcf = src4.reshape(NC, NS, EPT)
    u = _sc_agg(feat, srcf, dst4, ex, zeros)
    d2 = den.reshape(NC, DNR * D)[:, :N]
    return u, d2


def kernel(h, edge_index1, edge_index2, dst_feat, e1, e2, cellID, clusters,
           alpha0, beta0, gamma0, dt,
           W1, attn_l1, attn_r1, b1, W2, attn_l2, attn_r2, b2, Wlin, blin):
    src1 = _pad_edges(edge_index1[0])
    dst1 = _pad_edges(edge_index1[1])
    src2 = _pad_edges(edge_index2[0])
    dst2 = _pad_edges(edge_index2[1])
    zeros = jnp.zeros((N, D), jnp.float32)

    feat1, elr1 = _tc_front(h, W1, attn_l1.reshape(1, D),
                            attn_r1.reshape(1, D))
    u1, d1 = _gat_layer(feat1, elr1, src1, dst1, zeros)
    feat2, elr2 = _tc_mid(u1, d1, b1.reshape(1, D), W2,
                          attn_l2.reshape(1, D), attn_r2.reshape(1, D))
    u2, d2 = _gat_layer(feat2, elr2, src2, dst2, zeros)

    wlin_pad = jnp.zeros((D, D), jnp.float32).at[:, :3].set(Wlin)
    blin_pad = jnp.zeros((1, D), jnp.float32).at[0, :3].set(blin)
    scal = jnp.stack([alpha0, beta0, gamma0, dt])
    res = _tc_head(u2, d2, b2.reshape(1, D), wlin_pad, blin_pad,
                   dst_feat, scal)

    us = dst_feat[:, 0]
    s = dst_feat[:, 1]
    return (res[0], res[1], res[2], res[3], res[4],
            e1, e2, us, s, cellID, clusters)
